# baseline (speedup 1.0000x reference)
import functools

import jax
import jax.numpy as jnp
from jax import lax
from jax.experimental import pallas as pl
from jax.experimental.pallas import tpu as pltpu

_BLOCK_ROWS = 10000


def _half_pipeline(types_ref, x_ref, w1_ref, b1_ref, g2_ref, b2_ref, o_ref):
    x = x_ref[...].astype(jnp.bfloat16)
    ht = lax.dot_general(w1_ref[...], x, (((0,), (1,)), ((), ())),
                         preferred_element_type=jnp.float32)
    th = jnp.tanh(ht + b1_ref[...]).astype(jnp.bfloat16)
    p = lax.dot_general(g2_ref[...], th, (((0,), (0,)), ((), ())),
                        preferred_element_type=jnp.float32)
    p = p + b2_ref[...]
    t = types_ref[0]
    out = jnp.where(t == 0, p[0:1, :], 0.0)
    for s in range(1, p.shape[0]):
        out = jnp.where(t == s, p[s:s + 1, :], out)
    o_ref[...] = out.reshape(1, 1, out.shape[1])


def _fused_mlp_kernel(ta_ref, tb_ref, xa_ref, xb_ref, w1_ref, b1_ref,
                      g2_ref, b2_ref, oa_ref, ob_ref):
    _half_pipeline(ta_ref, xa_ref, w1_ref, b1_ref, g2_ref, b2_ref, oa_ref)
    _half_pipeline(tb_ref, xb_ref, w1_ref, b1_ref, g2_ref, b2_ref, ob_ref)


def kernel(features, batch, atom_types, W1, b1, W2, b2):
    del batch
    N, D = features.shape
    S, _, H = W1.shape
    SH = S * H

    w1pack = jnp.transpose(W1, (1, 0, 2)).reshape(D, SH).astype(jnp.bfloat16)
    b1col = b1.reshape(SH, 1)
    spec = jnp.arange(SH, dtype=jnp.int32) // H
    onehot = (spec[:, None] == jnp.arange(S, dtype=jnp.int32)[None, :])
    g2 = (W2.reshape(SH, 1) * onehot).astype(jnp.bfloat16)
    b2col = b2.reshape(S, 1)

    nb = _BLOCK_ROWS
    half = N // 2
    assert half % nb == 0, (N, nb)
    grid_n = half // nb
    types3d = atom_types.astype(jnp.int32).reshape(2 * grid_n, 1, nb)

    out_a, out_b = pl.pallas_call(
        _fused_mlp_kernel,
        grid=(grid_n,),
        in_specs=[
            pl.BlockSpec((1, 1, nb), lambda i: (i, 0, 0)),
            pl.BlockSpec((1, 1, nb), lambda i, g=grid_n: (i + g, 0, 0)),
            pl.BlockSpec((nb, D), lambda i: (i, 0)),
            pl.BlockSpec((nb, D), lambda i, g=grid_n: (i + g, 0)),
            pl.BlockSpec((D, SH), lambda i: (0, 0)),
            pl.BlockSpec((SH, 1), lambda i: (0, 0)),
            pl.BlockSpec((SH, S), lambda i: (0, 0)),
            pl.BlockSpec((S, 1), lambda i: (0, 0)),
        ],
        out_specs=[
            pl.BlockSpec((1, 1, nb), lambda i: (i, 0, 0)),
            pl.BlockSpec((1, 1, nb), lambda i: (i, 0, 0)),
        ],
        out_shape=[
            jax.ShapeDtypeStruct((grid_n, 1, nb), jnp.float32),
            jax.ShapeDtypeStruct((grid_n, 1, nb), jnp.float32),
        ],
        compiler_params=pltpu.CompilerParams(
            dimension_semantics=("parallel",)),
    )(types3d, types3d, features, features, w1pack, b1col, g2, b2col)
    return jnp.concatenate(
        [out_a.reshape(half, 1), out_b.reshape(half, 1)], axis=0)


# speedup vs baseline: 1.0350x; 1.0350x over previous
import functools

import jax
import jax.numpy as jnp
from jax import lax
from jax.experimental import pallas as pl
from jax.experimental.pallas import tpu as pltpu

_CHUNK_ROWS = 10000
_NBUF = 2


def _chunk_compute(types_row, x, w1_ref, b1_ref, g2_ref, b2_ref):
    xb = x.astype(jnp.bfloat16)
    ht = lax.dot_general(w1_ref[...], xb, (((0,), (1,)), ((), ())),
                         preferred_element_type=jnp.float32)
    th = jnp.tanh(ht + b1_ref[...]).astype(jnp.bfloat16)
    p = lax.dot_general(g2_ref[...], th, (((0,), (0,)), ((), ())),
                        preferred_element_type=jnp.float32)
    p = p + b2_ref[...]
    out = jnp.where(types_row == 0, p[0:1, :], 0.0)
    for s in range(1, p.shape[0]):
        out = jnp.where(types_row == s, p[s:s + 1, :], out)
    return out


def _fused_mlp_kernel(types_ref, x_hbm, w1_ref, b1_ref, g2_ref, b2_ref,
                      o_ref, xbuf, sem):
    nchunks, _, nb = o_ref.shape

    def _copy(i, slot):
        return pltpu.make_async_copy(
            x_hbm.at[pl.ds(i * nb, nb), :], xbuf.at[slot], sem.at[slot])

    _copy(0, 0).start()
    for i in range(nchunks):
        slot = i % _NBUF
        if i + 1 < nchunks:
            _copy(i + 1, (i + 1) % _NBUF).start()
        _copy(i, slot).wait()
        out = _chunk_compute(types_ref[i], xbuf[slot], w1_ref, b1_ref,
                             g2_ref, b2_ref)
        o_ref[i] = out.reshape(1, nb)


def kernel(features, batch, atom_types, W1, b1, W2, b2):
    del batch
    N, D = features.shape
    S, _, H = W1.shape
    SH = S * H

    w1pack = jnp.transpose(W1, (1, 0, 2)).reshape(D, SH).astype(jnp.bfloat16)
    b1col = b1.reshape(SH, 1)
    spec = jnp.arange(SH, dtype=jnp.int32) // H
    onehot = (spec[:, None] == jnp.arange(S, dtype=jnp.int32)[None, :])
    g2 = (W2.reshape(SH, 1) * onehot).astype(jnp.bfloat16)
    b2col = b2.reshape(S, 1)

    nb = _CHUNK_ROWS
    assert N % nb == 0, (N, nb)
    nchunks = N // nb
    types3d = atom_types.astype(jnp.int32).reshape(nchunks, 1, nb)

    out3d = pl.pallas_call(
        _fused_mlp_kernel,
        grid=(1,),
        in_specs=[
            pl.BlockSpec((nchunks, 1, nb), lambda i: (0, 0, 0)),
            pl.BlockSpec(memory_space=pl.ANY),
            pl.BlockSpec((D, SH), lambda i: (0, 0)),
            pl.BlockSpec((SH, 1), lambda i: (0, 0)),
            pl.BlockSpec((SH, S), lambda i: (0, 0)),
            pl.BlockSpec((S, 1), lambda i: (0, 0)),
        ],
        out_specs=pl.BlockSpec((nchunks, 1, nb), lambda i: (0, 0, 0)),
        out_shape=jax.ShapeDtypeStruct((nchunks, 1, nb), jnp.float32),
        scratch_shapes=[
            pltpu.VMEM((_NBUF, nb, D), jnp.float32),
            pltpu.SemaphoreType.DMA((_NBUF,)),
        ],
        compiler_params=pltpu.CompilerParams(
            dimension_semantics=("arbitrary",)),
    )(types3d, features, w1pack, b1col, g2, b2col)
    return out3d.reshape(N, 1)


# speedup vs baseline: 1.1008x; 1.0636x over previous
import functools

import jax
import jax.numpy as jnp
from jax import lax
from jax.experimental import pallas as pl
from jax.experimental.pallas import tpu as pltpu

_CHUNK_ROWS = 5000
_NBUF = 3


def _chunk_compute(types_row, x, w1_ref, b1_ref, g2_ref, b2_ref):
    xb = x.astype(jnp.bfloat16)
    ht = lax.dot_general(w1_ref[...], xb, (((0,), (1,)), ((), ())),
                         preferred_element_type=jnp.float32)
    th = jnp.tanh(ht + b1_ref[...]).astype(jnp.bfloat16)
    p = lax.dot_general(g2_ref[...], th, (((0,), (0,)), ((), ())),
                        preferred_element_type=jnp.float32)
    p = p + b2_ref[...]
    out = jnp.where(types_row == 0, p[0:1, :], 0.0)
    for s in range(1, p.shape[0]):
        out = jnp.where(types_row == s, p[s:s + 1, :], out)
    return out


def _fused_mlp_kernel(types_ref, x_hbm, w1_ref, b1_ref, g2_ref, b2_ref,
                      o_ref, xbuf, sem):
    nchunks, nb = o_ref.shape

    def _copy(i, slot):
        return pltpu.make_async_copy(
            x_hbm.at[pl.ds(i * nb, nb), :], xbuf.at[slot], sem.at[slot])

    for j in range(min(_NBUF - 1, nchunks)):
        _copy(j, j).start()
    for i in range(nchunks):
        slot = i % _NBUF
        nxt = i + _NBUF - 1
        if nxt < nchunks:
            _copy(nxt, nxt % _NBUF).start()
        _copy(i, slot).wait()
        out = _chunk_compute(types_ref[i:i + 1, :], xbuf[slot], w1_ref,
                             b1_ref, g2_ref, b2_ref)
        o_ref[i:i + 1, :] = out


def kernel(features, batch, atom_types, W1, b1, W2, b2):
    del batch
    N, D = features.shape
    S, _, H = W1.shape
    SH = S * H

    w1pack = jnp.transpose(W1, (1, 0, 2)).reshape(D, SH).astype(jnp.bfloat16)
    b1col = b1.reshape(SH, 1)
    spec = jnp.arange(SH, dtype=jnp.int32) // H
    onehot = (spec[:, None] == jnp.arange(S, dtype=jnp.int32)[None, :])
    g2 = (W2.reshape(SH, 1) * onehot).astype(jnp.bfloat16)
    b2col = b2.reshape(S, 1)

    nb = _CHUNK_ROWS
    assert N % nb == 0, (N, nb)
    nchunks = N // nb
    types2d = atom_types.astype(jnp.int32).reshape(nchunks, nb)

    out2d = pl.pallas_call(
        _fused_mlp_kernel,
        grid=(1,),
        in_specs=[
            pl.BlockSpec((nchunks, nb), lambda i: (0, 0)),
            pl.BlockSpec(memory_space=pl.ANY),
            pl.BlockSpec((D, SH), lambda i: (0, 0)),
            pl.BlockSpec((SH, 1), lambda i: (0, 0)),
            pl.BlockSpec((SH, S), lambda i: (0, 0)),
            pl.BlockSpec((S, 1), lambda i: (0, 0)),
        ],
        out_specs=pl.BlockSpec((nchunks, nb), lambda i: (0, 0)),
        out_shape=jax.ShapeDtypeStruct((nchunks, nb), jnp.float32),
        scratch_shapes=[
            pltpu.VMEM((_NBUF, nb, D), jnp.float32),
            pltpu.SemaphoreType.DMA((_NBUF,)),
        ],
        compiler_params=pltpu.CompilerParams(
            dimension_semantics=("arbitrary",)),
    )(types2d, features, w1pack, b1col, g2, b2col)
    return out2d.reshape(N, 1)
